# baseline (speedup 1.0000x reference)
import jax
import jax.numpy as jnp
from jax import lax
from jax.experimental import pallas as pl
from jax.experimental.pallas import tpu as pltpu
from jax.experimental.pallas import tpu_sc as plsc

VOCAB = 1000000
VEC = 64
SEQ = 50
BATCH = 4096

NC = 2
NS = 16
NW = NC * NS
BPW = BATCH // NW
SCH = 5
NCHUNK = SEQ // SCH
NV = VEC // 16
NH = VEC // 32


def _cbow_body(text, w_vec, b_vec, table, out_hbm,
               idx_v, buf0, buf1, acc_v, w_v, b_v, out_v, sem0, sem1):
  cid = lax.axis_index("c")
  sid = lax.axis_index("s")
  wid = sid * NC + cid
  base = wid * BPW

  pltpu.sync_copy(text.at[:, pl.ds(base, BPW)], idx_v)
  pltpu.sync_copy(w_vec, w_v)
  pltpu.sync_copy(b_vec, b_v)

  w_regs = [w_v[pl.ds(k * 16, 16)] for k in range(NV)]
  bias_v = b_v[...]
  lane = lax.iota(jnp.int32, 16)
  zero = jnp.zeros((16,), jnp.float32)

  def hsum(x):
    for sh in (8, 4, 2, 1):
      x = x + x.at[lane ^ sh].get(mode="promise_in_bounds")
    return x

  def zbody(c, carry):
    for k in range(NV):
      acc_v[c, pl.ds(k * 16, 16)] = zero
    return carry

  lax.fori_loop(0, BPW, zbody, 0)

  def start(ci, buf, sem):
    for j in range(SCH):
      pltpu.async_copy(table.at[idx_v.at[ci * SCH + j]], buf.at[j], sem)

  def wait(buf, sem):
    for s in range(SCH):
      pltpu.make_async_copy(table.at[pl.ds(0, BPW)], buf.at[s], sem).wait()

  def accumulate(buf):
    def body(c, carry):
      accs = [zero] * NV
      for s in range(SCH):
        for h in range(NH):
          packed = buf[s, c, pl.ds(h * 32, 32)]
          ev, od = plsc.unpack(packed, format=plsc.PackFormat.INTERLEAVED)
          accs[2 * h] = accs[2 * h] + ev
          accs[2 * h + 1] = accs[2 * h + 1] + od
      for k in range(NV):
        plsc.addupdate(acc_v.at[c, pl.ds(k * 16, 16)], accs[k])
      return carry
    lax.fori_loop(0, BPW, body, 0)

  start(0, buf0, sem0)

  def outer(gg, carry):
    start(2 * gg + 1, buf1, sem1)
    wait(buf0, sem0)
    accumulate(buf0)

    @pl.when(gg < NCHUNK // 2 - 1)
    def _():
      start(2 * gg + 2, buf0, sem0)

    wait(buf1, sem1)
    accumulate(buf1)
    return carry

  lax.fori_loop(0, NCHUNK // 2, outer, 0)

  def fgroup(g, carry):
    ovec = zero
    for j in range(16):
      c = g * 16 + j
      accs = [acc_v[c, pl.ds(k * 16, 16)] for k in range(NV)]
      p = jnp.maximum(accs[0], 0.0) * w_regs[0]
      for k in range(1, NV):
        p = p + jnp.maximum(accs[k], 0.0) * w_regs[k]
      total = hsum(p) + bias_v
      ovec = jnp.where(lane == j, total, ovec)
    out_v[pl.ds(g * 16, 16)] = ovec
    return carry

  lax.fori_loop(0, BPW // 16, fgroup, 0)

  pltpu.sync_copy(out_v, out_hbm.at[pl.ds(base, BPW)])


def kernel(text, W, w_lin, b_lin):
  w16 = W.astype(jnp.bfloat16)
  w16f = lax.optimization_barrier(w16.reshape(-1))
  table = w16f.reshape(VOCAB, VEC)

  w64 = w_lin.reshape(VEC)
  wperm = jnp.concatenate(
      [w64[0:32:2], w64[1:32:2], w64[32:64:2], w64[33:64:2]])
  b16 = jnp.broadcast_to(b_lin, (16,))

  mesh = plsc.VectorSubcoreMesh(core_axis_name="c", subcore_axis_name="s")
  kern = pl.kernel(
      _cbow_body,
      mesh=mesh,
      compiler_params=pltpu.CompilerParams(
          use_tc_tiling_on_sc=False, needs_layout_passes=False),
      out_type=jax.ShapeDtypeStruct((BATCH,), jnp.float32),
      scratch_types=[
          pltpu.VMEM((SEQ, BPW), jnp.int32),
          pltpu.VMEM((SCH, BPW, VEC), jnp.bfloat16),
          pltpu.VMEM((SCH, BPW, VEC), jnp.bfloat16),
          pltpu.VMEM((BPW, VEC), jnp.float32),
          pltpu.VMEM((VEC,), jnp.float32),
          pltpu.VMEM((16,), jnp.float32),
          pltpu.VMEM((BPW,), jnp.float32),
          pltpu.SemaphoreType.DMA,
          pltpu.SemaphoreType.DMA,
      ],
  )
  return kern(text, wperm, b16, table)


# speedup vs baseline: 1.2332x; 1.2332x over previous
import jax
import jax.numpy as jnp
from jax import lax
from jax.experimental import pallas as pl
from jax.experimental.pallas import tpu as pltpu
from jax.experimental.pallas import tpu_sc as plsc

VOCAB = 1000000
VEC = 64
SEQ = 50
BATCH = 4096

NC = 2
NS = 16
NW = NC * NS
BPW = BATCH // NW
SCH = 5
NCHUNK = SEQ // SCH
NV = VEC // 16
NH = VEC // 32


def _cbow_body(text, w_vec, b_vec, table, out_hbm,
               idx_v, buf0, buf1, acc_v, w_v, b_v, out_v, sem0, sem1):
  cid = lax.axis_index("c")
  sid = lax.axis_index("s")
  wid = sid * NC + cid
  base = wid * BPW

  pltpu.sync_copy(text.at[:, pl.ds(base, BPW)], idx_v)
  pltpu.sync_copy(w_vec, w_v)
  pltpu.sync_copy(b_vec, b_v)

  w_regs = [w_v[pl.ds(k * 16, 16)] for k in range(NV)]
  bias_v = b_v[...]
  lane = lax.iota(jnp.int32, 16)
  zero = jnp.zeros((16,), jnp.float32)

  def hsum(x):
    for sh in (8, 4, 2, 1):
      x = x + x.at[lane ^ sh].get(mode="promise_in_bounds")
    return x

  def zbody(c, carry):
    for k in range(NV):
      acc_v[c, pl.ds(k * 16, 16)] = zero
    return carry

  lax.fori_loop(0, BPW, zbody, 0)

  def start(ci, buf, sem):
    for j in range(SCH):
      pltpu.async_copy(table.at[idx_v.at[ci * SCH + j]], buf.at[j], sem)

  def wait(buf, sem):
    for s in range(SCH):
      pltpu.make_async_copy(table.at[pl.ds(0, BPW)], buf.at[s], sem).wait()

  def accumulate(buf):
    def body(c, carry):
      accs = [zero] * NV
      for s in range(SCH):
        for h in range(NH):
          packed = buf[s, c, pl.ds(h * 32, 32)]
          ev, od = plsc.unpack(packed, format=plsc.PackFormat.INTERLEAVED)
          accs[2 * h] = accs[2 * h] + ev
          accs[2 * h + 1] = accs[2 * h + 1] + od
      for k in range(NV):
        plsc.addupdate(acc_v.at[c, pl.ds(k * 16, 16)], accs[k])
      return carry
    lax.fori_loop(0, BPW, body, 0)

  start(0, buf0, sem0)

  def outer(gg, carry):
    start(2 * gg + 1, buf1, sem1)
    wait(buf0, sem0)
    accumulate(buf0)

    @pl.when(gg < NCHUNK // 2 - 1)
    def _():
      start(2 * gg + 2, buf0, sem0)

    wait(buf1, sem1)
    accumulate(buf1)
    return carry

  lax.fori_loop(0, NCHUNK // 2, outer, 0)

  def fgroup(g, carry):
    ovec = zero
    for j in range(16):
      c = g * 16 + j
      accs = [acc_v[c, pl.ds(k * 16, 16)] for k in range(NV)]
      p = jnp.maximum(accs[0], 0.0) * w_regs[0]
      for k in range(1, NV):
        p = p + jnp.maximum(accs[k], 0.0) * w_regs[k]
      total = hsum(p) + bias_v
      ovec = jnp.where(lane == j, total, ovec)
    out_v[pl.ds(g * 16, 16)] = ovec
    return carry

  lax.fori_loop(0, BPW // 16, fgroup, 0)

  pltpu.sync_copy(out_v, out_hbm.at[pl.ds(base, BPW)])


def kernel(text, W, w_lin, b_lin):
  table = W.astype(jnp.bfloat16)

  w64 = w_lin.reshape(VEC)
  wperm = jnp.concatenate(
      [w64[0:32:2], w64[1:32:2], w64[32:64:2], w64[33:64:2]])
  b16 = jnp.broadcast_to(b_lin, (16,))

  mesh = plsc.VectorSubcoreMesh(core_axis_name="c", subcore_axis_name="s")
  kern = pl.kernel(
      _cbow_body,
      mesh=mesh,
      compiler_params=pltpu.CompilerParams(
          use_tc_tiling_on_sc=False, needs_layout_passes=False),
      out_type=jax.ShapeDtypeStruct((BATCH,), jnp.float32),
      scratch_types=[
          pltpu.VMEM((SEQ, BPW), jnp.int32),
          pltpu.VMEM((SCH, BPW, VEC), jnp.bfloat16),
          pltpu.VMEM((SCH, BPW, VEC), jnp.bfloat16),
          pltpu.VMEM((BPW, VEC), jnp.float32),
          pltpu.VMEM((VEC,), jnp.float32),
          pltpu.VMEM((16,), jnp.float32),
          pltpu.VMEM((BPW,), jnp.float32),
          pltpu.SemaphoreType.DMA,
          pltpu.SemaphoreType.DMA,
      ],
  )
  return kern(text, wperm, b16, table)


# speedup vs baseline: 1.6411x; 1.3307x over previous
import jax
import jax.numpy as jnp
from jax import lax
from jax.experimental import pallas as pl
from jax.experimental.pallas import tpu as pltpu
from jax.experimental.pallas import tpu_sc as plsc

VOCAB = 1000000
VEC = 64
SEQ = 50
BATCH = 4096

NC = 2
NS = 16
NW = NC * NS
BPW = BATCH // NW
SCH = 5
NCHUNK = SEQ // SCH
NV = VEC // 16


def _cbow_body(text, w_vec, b_vec, table, out_hbm,
               idx_v, buf0, buf1, acc_v, w_v, b_v, out_v, sem0, sem1):
  cid = lax.axis_index("c")
  sid = lax.axis_index("s")
  wid = sid * NC + cid
  base = wid * BPW

  pltpu.sync_copy(text.at[:, pl.ds(base, BPW)], idx_v)
  pltpu.sync_copy(w_vec, w_v)
  pltpu.sync_copy(b_vec, b_v)

  w_regs = [w_v[pl.ds(k * 16, 16)] for k in range(NV)]
  bias_v = b_v[...]
  lane = lax.iota(jnp.int32, 16)
  zero = jnp.zeros((16,), jnp.float32)

  def hsum(x):
    for sh in (8, 4, 2, 1):
      x = x + x.at[lane ^ sh].get(mode="promise_in_bounds")
    return x

  def zbody(c, carry):
    for k in range(NV):
      acc_v[c, pl.ds(k * 16, 16)] = zero
    return carry

  lax.fori_loop(0, BPW, zbody, 0)

  def start(ci, buf, sem):
    for j in range(SCH):
      pltpu.async_copy(table.at[idx_v.at[ci * SCH + j]], buf.at[j], sem)

  def wait(buf, sem):
    for s in range(SCH):
      pltpu.make_async_copy(table.at[pl.ds(0, BPW)], buf.at[s], sem).wait()

  def accumulate(buf):
    def body(c, carry):
      for k in range(NV):
        v = buf[0, c, pl.ds(k * 16, 16)]
        for s in range(1, SCH):
          v = v + buf[s, c, pl.ds(k * 16, 16)]
        plsc.addupdate(acc_v.at[c, pl.ds(k * 16, 16)], v)
      return carry
    lax.fori_loop(0, BPW, body, 0)

  start(0, buf0, sem0)

  def outer(gg, carry):
    start(2 * gg + 1, buf1, sem1)
    wait(buf0, sem0)
    accumulate(buf0)

    @pl.when(gg < NCHUNK // 2 - 1)
    def _():
      start(2 * gg + 2, buf0, sem0)

    wait(buf1, sem1)
    accumulate(buf1)
    return carry

  lax.fori_loop(0, NCHUNK // 2, outer, 0)

  def fgroup(g, carry):
    ovec = zero
    for j in range(16):
      c = g * 16 + j
      accs = [acc_v[c, pl.ds(k * 16, 16)] for k in range(NV)]
      p = jnp.maximum(accs[0], 0.0) * w_regs[0]
      for k in range(1, NV):
        p = p + jnp.maximum(accs[k], 0.0) * w_regs[k]
      total = hsum(p) + bias_v
      ovec = jnp.where(lane == j, total, ovec)
    out_v[pl.ds(g * 16, 16)] = ovec
    return carry

  lax.fori_loop(0, BPW // 16, fgroup, 0)

  pltpu.sync_copy(out_v, out_hbm.at[pl.ds(base, BPW)])


def kernel(text, W, w_lin, b_lin):
  w64 = w_lin.reshape(VEC)
  b16 = jnp.broadcast_to(b_lin, (16,))

  mesh = plsc.VectorSubcoreMesh(core_axis_name="c", subcore_axis_name="s")
  kern = pl.kernel(
      _cbow_body,
      mesh=mesh,
      compiler_params=pltpu.CompilerParams(use_tc_tiling_on_sc=False),
      out_type=jax.ShapeDtypeStruct((BATCH,), jnp.float32),
      scratch_types=[
          pltpu.VMEM((SEQ, BPW), jnp.int32),
          pltpu.VMEM((SCH, BPW, VEC), jnp.float32),
          pltpu.VMEM((SCH, BPW, VEC), jnp.float32),
          pltpu.VMEM((BPW, VEC), jnp.float32),
          pltpu.VMEM((VEC,), jnp.float32),
          pltpu.VMEM((16,), jnp.float32),
          pltpu.VMEM((BPW,), jnp.float32),
          pltpu.SemaphoreType.DMA,
          pltpu.SemaphoreType.DMA,
      ],
  )
  return kern(text, w64, b16, W)


# speedup vs baseline: 1.6424x; 1.0008x over previous
import jax
import jax.numpy as jnp
from jax import lax
from jax.experimental import pallas as pl
from jax.experimental.pallas import tpu as pltpu
from jax.experimental.pallas import tpu_sc as plsc

VOCAB = 1000000
VEC = 64
SEQ = 50
BATCH = 4096

NC = 2
NS = 16
NW = NC * NS
BPW = BATCH // NW
SCH = 5
NCHUNK = SEQ // SCH
NV = VEC // 16


def _cbow_body(text, w_vec, b_vec, table, out_hbm,
               idx_v, buf0, buf1, acc_v, w_v, b_v, out_v, sem0, sem1):
  cid = lax.axis_index("c")
  sid = lax.axis_index("s")
  wid = sid * NC + cid
  base = wid * BPW

  pltpu.sync_copy(text.at[:, pl.ds(base, BPW)], idx_v)
  pltpu.sync_copy(w_vec, w_v)
  pltpu.sync_copy(b_vec, b_v)

  w_regs = [w_v[pl.ds(k * 16, 16)] for k in range(NV)]
  bias_v = b_v[...]
  lane = lax.iota(jnp.int32, 16)
  zero = jnp.zeros((16,), jnp.float32)

  def hsum(x):
    for sh in (8, 4, 2, 1):
      x = x + x.at[lane ^ sh].get(mode="promise_in_bounds")
    return x

  def zbody(c, carry):
    for k in range(NV):
      acc_v[c, pl.ds(k * 16, 16)] = zero
    return carry

  lax.fori_loop(0, BPW, zbody, 0)

  def start(ci, buf, sem):
    for j in range(SCH):
      pltpu.async_copy(table.at[idx_v.at[ci * SCH + j]], buf.at[j], sem)

  def wait(buf, sem):
    for s in range(SCH):
      pltpu.make_async_copy(table.at[pl.ds(0, BPW)], buf.at[s], sem).wait()

  def accumulate(buf):
    def body(c, carry):
      for k in range(NV):
        v = buf[0, c, pl.ds(k * 16, 16)]
        for s in range(1, SCH):
          v = v + buf[s, c, pl.ds(k * 16, 16)]
        plsc.addupdate(acc_v.at[c, pl.ds(k * 16, 16)], v)
      return carry
    lax.fori_loop(0, BPW, body, 0)

  start(0, buf0, sem0)

  def outer(gg, carry):
    start(2 * gg + 1, buf1, sem1)
    wait(buf0, sem0)
    accumulate(buf0)

    @pl.when(gg < NCHUNK // 2 - 1)
    def _():
      start(2 * gg + 2, buf0, sem0)

    wait(buf1, sem1)
    accumulate(buf1)
    return carry

  lax.fori_loop(0, NCHUNK // 2, outer, 0)

  def fgroup(g, carry):
    ovec = zero
    for j in range(16):
      c = g * 16 + j
      accs = [acc_v[c, pl.ds(k * 16, 16)] for k in range(NV)]
      p = jnp.maximum(accs[0], 0.0) * w_regs[0]
      for k in range(1, NV):
        p = p + jnp.maximum(accs[k], 0.0) * w_regs[k]
      total = hsum(p) + bias_v
      ovec = jnp.where(lane == j, total, ovec)
    out_v[pl.ds(g * 16, 16)] = ovec
    return carry

  lax.fori_loop(0, BPW // 16, fgroup, 0)

  pltpu.sync_copy(out_v, out_hbm.at[pl.ds(base, BPW)])


def kernel(text, W, w_lin, b_lin):
  w64 = w_lin.reshape(VEC)
  b16 = jnp.broadcast_to(b_lin, (16,))

  mesh = plsc.VectorSubcoreMesh(core_axis_name="c", subcore_axis_name="s")
  kern = pl.kernel(
      _cbow_body,
      mesh=mesh,
      compiler_params=pltpu.CompilerParams(
          use_tc_tiling_on_sc=False, skip_device_barrier=True),
      out_type=jax.ShapeDtypeStruct((BATCH,), jnp.float32),
      scratch_types=[
          pltpu.VMEM((SEQ, BPW), jnp.int32),
          pltpu.VMEM((SCH, BPW, VEC), jnp.float32),
          pltpu.VMEM((SCH, BPW, VEC), jnp.float32),
          pltpu.VMEM((BPW, VEC), jnp.float32),
          pltpu.VMEM((VEC,), jnp.float32),
          pltpu.VMEM((16,), jnp.float32),
          pltpu.VMEM((BPW,), jnp.float32),
          pltpu.SemaphoreType.DMA,
          pltpu.SemaphoreType.DMA,
      ],
  )
  return kern(text, w64, b16, W)


# speedup vs baseline: 1.6431x; 1.0004x over previous
import jax
import jax.numpy as jnp
from jax import lax
from jax.experimental import pallas as pl
from jax.experimental.pallas import tpu as pltpu
from jax.experimental.pallas import tpu_sc as plsc

VOCAB = 1000000
VEC = 64
SEQ = 50
BATCH = 4096

NC = 2
NS = 16
NW = NC * NS
BPW = BATCH // NW
SCH = 5
NCHUNK = SEQ // SCH
NV = VEC // 16


def _cbow_body(text, w_vec, b_vec, table, out_hbm,
               idx_v, buf0, buf1, acc_v, w_v, b_v, out_v, sem0, sem1):
  cid = lax.axis_index("c")
  sid = lax.axis_index("s")
  wid = sid * NC + cid
  base = wid * BPW

  pltpu.sync_copy(text.at[:, pl.ds(base, BPW)], idx_v)
  pltpu.sync_copy(w_vec, w_v)
  pltpu.sync_copy(b_vec, b_v)

  w_regs = [w_v[pl.ds(k * 16, 16)] for k in range(NV)]
  bias_v = b_v[...]
  lane = lax.iota(jnp.int32, 16)
  zero = jnp.zeros((16,), jnp.float32)

  def hsum(x):
    for sh in (8, 4, 2, 1):
      x = x + x.at[lane ^ sh].get(mode="promise_in_bounds")
    return x

  def zbody(c, carry):
    for k in range(NV):
      acc_v[c, pl.ds(k * 16, 16)] = zero
    return carry

  lax.fori_loop(0, BPW, zbody, 0)

  def start(ci, buf, sem):
    for j in range(SCH):
      pltpu.async_copy(table.at[idx_v.at[ci * SCH + j]], buf.at[j], sem)

  def wait(buf, sem):
    for s in range(SCH):
      pltpu.make_async_copy(table.at[pl.ds(0, BPW)], buf.at[s], sem).wait()

  def accumulate(buf):
    def body(c, carry):
      for k in range(NV):
        v = buf[0, c, pl.ds(k * 16, 16)]
        for s in range(1, SCH):
          v = v + buf[s, c, pl.ds(k * 16, 16)]
        plsc.addupdate(acc_v.at[c, pl.ds(k * 16, 16)], v)
      return carry
    lax.fori_loop(0, BPW, body, 0)

  start(0, buf0, sem0)

  def outer(gg, carry):
    start(2 * gg + 1, buf1, sem1)
    wait(buf0, sem0)
    accumulate(buf0)

    @pl.when(gg < NCHUNK // 2 - 1)
    def _():
      start(2 * gg + 2, buf0, sem0)

    wait(buf1, sem1)
    accumulate(buf1)
    return carry

  lax.fori_loop(0, NCHUNK // 2, outer, 0)

  def fgroup(g, carry):
    ovec = zero
    for j in range(16):
      c = g * 16 + j
      accs = [acc_v[c, pl.ds(k * 16, 16)] for k in range(NV)]
      p = jnp.maximum(accs[0], 0.0) * w_regs[0]
      for k in range(1, NV):
        p = p + jnp.maximum(accs[k], 0.0) * w_regs[k]
      total = hsum(p) + bias_v
      ovec = jnp.where(lane == j, total, ovec)
    out_v[pl.ds(g * 16, 16)] = ovec
    return carry

  lax.fori_loop(0, BPW // 16, fgroup, 0)

  pltpu.sync_copy(out_v, out_hbm.at[pl.ds(base, BPW)])


def kernel(text, W, w_lin, b_lin):
  w64 = w_lin.reshape(VEC)
  b16 = jnp.broadcast_to(b_lin, (16,))

  mesh = plsc.VectorSubcoreMesh(core_axis_name="c", subcore_axis_name="s")
  kern = pl.kernel(
      _cbow_body,
      mesh=mesh,
      compiler_params=pltpu.CompilerParams(use_tc_tiling_on_sc=False),
      out_type=jax.ShapeDtypeStruct((BATCH,), jnp.float32),
      scratch_types=[
          pltpu.VMEM((SEQ, BPW), jnp.int32),
          pltpu.VMEM((SCH, BPW, VEC), jnp.float32),
          pltpu.VMEM((SCH, BPW, VEC), jnp.float32),
          pltpu.VMEM((BPW, VEC), jnp.float32),
          pltpu.VMEM((VEC,), jnp.float32),
          pltpu.VMEM((16,), jnp.float32),
          pltpu.VMEM((BPW,), jnp.float32),
          pltpu.SemaphoreType.DMA,
          pltpu.SemaphoreType.DMA,
      ],
  )
  return kern(text, w64, b16, W)
